# baseline (speedup 1.0000x reference)
import functools

import jax
import jax.numpy as jnp
from jax import lax
from jax.experimental import pallas as pl
from jax.experimental.pallas import tpu as pltpu
from jax.experimental.pallas import tpu_sc as plsc

N = 10000
E = 320000
B = 64
D = 128
C = 10
NBLK = 79
PAD_N = NBLK * 128
NW = 32
EPAD = 327680
EPW = EPAD // NW
CH = EPW // 128
RPT = PAD_N // 16

_f32 = jnp.float32

_SC_MESH = plsc.VectorSubcoreMesh(core_axis_name="c", subcore_axis_name="s",
                                  num_cores=2, num_subcores=16)


def _sc_scalar_body(tab_hbm, src_hbm, dst_hbm, out_hbm, tab_v, src_v, dst_v,
                    acc_v):
    c = lax.axis_index("c")
    s = lax.axis_index("s")
    wid = s * 2 + c
    pltpu.sync_copy(tab_hbm, tab_v)
    pltpu.sync_copy(src_hbm.at[pl.ds(wid * EPW, EPW)], src_v)
    pltpu.sync_copy(dst_hbm.at[pl.ds(wid * EPW, EPW)], dst_v)

    def zero(i, carry):
        acc_v[pl.ds(i * 16, 16)] = jnp.zeros((16,), _f32)
        return carry

    lax.fori_loop(0, PAD_N // 16, zero, 0)

    def edge(i, carry):
        si = src_v[pl.ds(i * 16, 16)]
        di = dst_v[pl.ds(i * 16, 16)]
        vals = plsc.load_gather(tab_v, [si])
        plsc.addupdate_scatter(acc_v, [di], vals)
        return carry

    lax.fori_loop(0, EPW // 16, edge, 0)
    pltpu.sync_copy(acc_v, out_hbm.at[pl.ds(wid * PAD_N, PAD_N)])


_sc_scalar_scatter = pl.kernel(
    _sc_scalar_body,
    out_type=jax.ShapeDtypeStruct((NW * PAD_N,), _f32),
    mesh=_SC_MESH,
    scratch_types=[
        pltpu.VMEM((PAD_N,), _f32),
        pltpu.VMEM((EPW,), jnp.int32),
        pltpu.VMEM((EPW,), jnp.int32),
        pltpu.VMEM((PAD_N,), _f32),
    ],
    compiler_params=pltpu.CompilerParams(needs_layout_passes=False),
    name="sc_scalar_scatter",
)


def _sc_feat_body(a_hbm, src_hbm, dst_hbm, out_hbm, src_v, dst_v, rows0,
                  shacc, sem0):
    c = lax.axis_index("c")
    s = lax.axis_index("s")
    wid = s * 2 + c
    def zrow(k, carry):
        rows0[k // 8, pl.ds((k % 8) * 16, 16)] = jnp.zeros((16,), _f32)
        return carry

    lax.fori_loop(0, 256 * 8, zrow, 0)
    base = s * RPT
    for t in range(2):
        pltpu.sync_copy(rows0, shacc.at[pl.ds(base + t * 256, 256)])
    pltpu.sync_copy(rows0.at[pl.ds(0, RPT - 512)],
                    shacc.at[pl.ds(base + 512, RPT - 512)])
    plsc.subcore_barrier()

    hch = CH // 4
    for half in range(2):
        pltpu.sync_copy(src_hbm.at[pl.ds(wid * 2 * hch + half * hch, hch)],
                        src_v)
        pltpu.sync_copy(
            dst_hbm.at[pl.ds(wid * 4 * hch + half * 2 * hch, 2 * hch)],
            dst_v)

        def chunk(j, carry):
            pltpu.async_copy(a_hbm.at[src_v.at[j, 0]], rows0, sem0).wait()
            pltpu.sync_copy(rows0.at[pl.ds(0, 128)],
                            shacc.at[dst_v.at[2 * j, 0]], add=True)
            pltpu.sync_copy(rows0.at[pl.ds(128, 128)],
                            shacc.at[dst_v.at[2 * j + 1, 0]], add=True)
            return carry

        lax.fori_loop(0, hch, chunk, 0)

    plsc.subcore_barrier()
    pltpu.sync_copy(shacc.at[pl.ds(base, RPT)],
                    out_hbm.at[c].at[pl.ds(base, RPT)])


_sc_feat_scatter = pl.kernel(
    _sc_feat_body,
    out_type=jax.ShapeDtypeStruct((2, PAD_N, D), _f32),
    mesh=_SC_MESH,
    scratch_types=[
        pltpu.VMEM((CH // 4, 1, 256), jnp.int32),
        pltpu.VMEM((CH // 2, 1, 128), jnp.int32),
        pltpu.VMEM((256, D), _f32),
        pltpu.VMEM_SHARED((PAD_N, D), _f32),
        pltpu.SemaphoreType.DMA,
    ],
    compiler_params=pltpu.CompilerParams(needs_layout_passes=False),
    name="sc_feat_scatter",
)


def _tcol(row, eye):
    return lax.dot_general(eye, row, (((1,), (1,)), ((), ())),
                           preferred_element_type=_f32)


def _trow(col, eye):
    return lax.dot_general(col, eye, (((0,), (0,)), ((), ())),
                           preferred_element_type=_f32)


def _mm_scale_body(x_ref, w_ref, s_ref, m_ref, aggp_ref, a_ref, dinv_ref):
    eye = jnp.eye(128, dtype=_f32)
    w = w_ref[...]

    acc = lax.fori_loop(0, NW, lambda p, a: a + aggp_ref[p],
                        jnp.zeros((NBLK, 128), _f32))
    dinv = m_ref[...] * lax.rsqrt(acc + 1.0)
    dinv_ref[...] = dinv

    def blk(i, carry):
        xb = x_ref[pl.ds(i * 128, 128), :]
        s_col = _tcol(s_ref[pl.ds(i, 1), :], eye)
        d_col = _tcol(dinv_ref[pl.ds(i, 1), :], eye)
        xw = jnp.dot(xb * s_col, w, preferred_element_type=_f32)
        a_ref[pl.ds(i * 128, 128), :] = xw * d_col
        return carry

    lax.fori_loop(0, NBLK, blk, 0)


def _tc_mm_scale(x, w, s_row, m_row, aggp):
    return pl.pallas_call(
        _mm_scale_body,
        out_shape=(jax.ShapeDtypeStruct((PAD_N, D), _f32),
                   jax.ShapeDtypeStruct((NBLK, 128), _f32)),
    )(x, w, s_row, m_row, aggp)


def _post_conv_body(sc_ref, a_ref, dinv_ref, b_ref, wrel_ref, wroot_ref,
                    m_ref, h_ref, v_ref, t_ref):
    eye = jnp.eye(128, dtype=_f32)
    b = b_ref[...]
    wrel_c = _tcol(wrel_ref[...], eye)
    wroot_c = _tcol(wroot_ref[...], eye)

    def blk(i, carry):
        d_col = _tcol(dinv_ref[pl.ds(i, 1), :], eye)
        tot = (sc_ref[0, pl.ds(i * 128, 128), :]
               + sc_ref[1, pl.ds(i * 128, 128), :]
               + a_ref[pl.ds(i * 128, 128), :])
        hb = jnp.maximum(d_col * tot + b, 0.0)
        h_ref[pl.ds(i * 128, 128), :] = hb
        r_col = jnp.dot(hb, wrel_c, preferred_element_type=_f32)
        t_col = jnp.dot(hb, wroot_c, preferred_element_type=_f32)
        v_ref[pl.ds(i, 1), :] = _trow(r_col, eye) * m_ref[pl.ds(i, 1), :]
        t_ref[pl.ds(i, 1), :] = _trow(t_col, eye)
        return carry

    lax.fori_loop(0, NBLK, blk, 0)


def _tc_post_conv(sc, a, dinv_row, b_row, wrel_row, wroot_row, m_row):
    return pl.pallas_call(
        _post_conv_body,
        out_shape=(jax.ShapeDtypeStruct((PAD_N, D), _f32),
                   jax.ShapeDtypeStruct((NBLK, 128), _f32),
                   jax.ShapeDtypeStruct((NBLK, 128), _f32)),
    )(sc, a, dinv_row, b_row, wrel_row, wroot_row, m_row)


def _score_body(aggp_ref, t_ref, m_ref, pb_ref, score_ref):
    acc = lax.fori_loop(0, NW, lambda p, a: a + aggp_ref[p],
                        jnp.zeros((NBLK, 128), _f32))
    score_ref[...] = jnp.tanh(m_ref[...] * acc + t_ref[...] + pb_ref[0, 0])


def _tc_score(aggp, t_row, m_row, pb):
    return pl.pallas_call(
        _score_body,
        in_specs=[
            pl.BlockSpec(memory_space=pltpu.VMEM),
            pl.BlockSpec(memory_space=pltpu.VMEM),
            pl.BlockSpec(memory_space=pltpu.VMEM),
            pl.BlockSpec(memory_space=pltpu.SMEM),
        ],
        out_shape=jax.ShapeDtypeStruct((NBLK, 128), _f32),
    )(aggp, t_row, m_row, pb)


def _topk_body(s2_ref, s1_ref, bf_ref, mp_ref, fb_ref, kept_ref, srow_ref,
               rank_ref):
    eye = jnp.eye(128, dtype=_f32)
    bf = bf_ref[...]
    mp = mp_ref[...]

    kn = jnp.zeros((NBLK, 128), _f32)
    for g in range(B):
        cnt = jnp.sum(jnp.where(bf == g, mp, 0.0))
        kg = jnp.floor((cnt + 1.0) * 0.5)
        kn = kn + jnp.where(bf == g, kg, 0.0)

    rank_ref[...] = jnp.zeros((NBLK, 128), _f32)
    lane = lax.broadcasted_iota(jnp.int32, (1, 128), 1).astype(_f32)
    sub = lax.broadcasted_iota(jnp.int32, (128, 1), 0).astype(_f32)

    def jloop(j, carry):
        s2_jc = _tcol(s2_ref[pl.ds(j, 1), :], eye)
        s1_jc = _tcol(s1_ref[pl.ds(j, 1), :], eye)
        bf_jc = _tcol(bf_ref[pl.ds(j, 1), :], eye)
        mp_jc = _tcol(mp_ref[pl.ds(j, 1), :], eye)
        idx_j = sub + jnp.float32(128.0) * j.astype(_f32)
        fbj = fb_ref[j]
        fbj1 = fb_ref[j + 1]

        def iloop(i, carry2):
            fbi = fb_ref[i]
            fbi1 = fb_ref[i + 1]

            @pl.when(jnp.logical_and(fbj <= fbi1, fbj1 >= fbi))
            def _():
                s2_i = s2_ref[pl.ds(i, 1), :]
                s1_i = s1_ref[pl.ds(i, 1), :]
                bf_i = bf_ref[pl.ds(i, 1), :]
                idx_i = lane + jnp.float32(128.0) * i.astype(_f32)
                eq2 = s2_jc == s2_i
                beats = (s2_jc > s2_i) | (eq2 & ((s1_jc > s1_i) | (
                    (s1_jc == s1_i) & (idx_j < idx_i))))
                same = (bf_jc == bf_i) & (mp_jc > 0.0)
                contrib = jnp.sum(jnp.where(beats & same, 1.0, 0.0),
                                  axis=0, keepdims=True)
                rank_ref[pl.ds(i, 1), :] += contrib

            return carry2

        lax.fori_loop(0, NBLK, iloop, 0)
        return carry

    lax.fori_loop(0, NBLK, jloop, 0)

    kept = jnp.where((rank_ref[...] < kn) & (mp > 0.0) & (bf < float(B)),
                     1.0, 0.0)
    kept_ref[...] = kept
    srow_ref[...] = s2_ref[...] * kept


def _tc_topk(s2_row, s1_row, bf_row, mp_row, fb):
    return pl.pallas_call(
        _topk_body,
        in_specs=[
            pl.BlockSpec(memory_space=pltpu.VMEM),
            pl.BlockSpec(memory_space=pltpu.VMEM),
            pl.BlockSpec(memory_space=pltpu.VMEM),
            pl.BlockSpec(memory_space=pltpu.VMEM),
            pl.BlockSpec(memory_space=pltpu.SMEM),
        ],
        out_shape=(jax.ShapeDtypeStruct((NBLK, 128), _f32),
                   jax.ShapeDtypeStruct((NBLK, 128), _f32)),
        scratch_shapes=[pltpu.VMEM((NBLK, 128), _f32)],
    )(s2_row, s1_row, bf_row, mp_row, fb)


def _readout_body(sc_ref, a_ref, dinv_ref, b3_ref, m2_ref, bf_ref, l1w_ref,
                  l1b_ref, l2w_ref, l2b_ref, out_ref):
    eye = jnp.eye(128, dtype=_f32)
    b3 = b3_ref[...]
    g_row = lax.broadcasted_iota(jnp.int32, (1, B), 1).astype(_f32)

    def blk(i, carry):
        pooled, cnt = carry
        d_col = _tcol(dinv_ref[pl.ds(i, 1), :], eye)
        tot = (sc_ref[0, pl.ds(i * 128, 128), :]
               + sc_ref[1, pl.ds(i * 128, 128), :]
               + a_ref[pl.ds(i * 128, 128), :])
        hb = jnp.maximum(d_col * tot + b3, 0.0)
        m_col = _tcol(m2_ref[pl.ds(i, 1), :], eye)
        b_col = _tcol(bf_ref[pl.ds(i, 1), :], eye)
        onehot = jnp.where(b_col == g_row, 1.0, 0.0)
        pooled = pooled + lax.dot_general(onehot, hb * m_col,
                                          (((0,), (0,)), ((), ())),
                                          preferred_element_type=_f32)
        cnt = cnt + lax.dot_general(onehot, m_col, (((0,), (0,)), ((), ())),
                                    preferred_element_type=_f32)
        return pooled, cnt

    pooled, cnt = lax.fori_loop(
        0, NBLK, blk,
        (jnp.zeros((B, 128), _f32), jnp.zeros((B, 1), _f32)))
    pooled = pooled / jnp.maximum(cnt, 1.0)
    z = jnp.maximum(jnp.dot(pooled, l1w_ref[...],
                            preferred_element_type=_f32) + l1b_ref[...], 0.0)
    logits = jnp.dot(z, l2w_ref[...], preferred_element_type=_f32) \
        + l2b_ref[...]
    valid = lax.broadcasted_iota(jnp.int32, (1, 128), 1) < C
    neg = jnp.where(valid, logits, -jnp.inf)
    mx = jnp.max(neg, axis=1, keepdims=True)
    ez = jnp.where(valid, jnp.exp(logits - mx), 0.0)
    lse = jnp.log(jnp.sum(ez, axis=1, keepdims=True))
    out_ref[...] = logits - mx - lse


def _tc_readout(sc, a, dinv_row, b3_row, m2_row, bf_row, l1w, l1b_row,
                l2w_pad, l2b_row):
    return pl.pallas_call(
        _readout_body,
        out_shape=jax.ShapeDtypeStruct((B, 128), _f32),
    )(sc, a, dinv_row, b3_row, m2_row, bf_row, l1w, l1b_row, l2w_pad,
      l2b_row)


def kernel(x, edge_index, batch, W1, b1, p1_Wrel, p1_Wroot, p1_b, W2, b2,
           p2_Wrel, p2_Wroot, p2_b, W3, b3, l1W, l1b, l2W, l2b):
    n = x.shape[0]

    x_pad = jnp.pad(x, ((0, PAD_N - n), (0, 0)))
    batch_pad = jnp.pad(batch, (0, PAD_N - n), constant_values=B)
    bf_row = batch_pad.astype(_f32).reshape(NBLK, 128)
    fb = jnp.concatenate(
        [batch_pad.reshape(NBLK, 128)[:, 0],
         jnp.full((1,), B, jnp.int32)])
    m0_row = (jnp.arange(PAD_N, dtype=jnp.int32) < n).astype(_f32) \
        .reshape(NBLK, 128)

    srcf = jnp.pad(edge_index[0], (0, EPAD - E), constant_values=n)
    dstf = jnp.pad(edge_index[1], (0, EPAD - E), constant_values=n)
    src2d = srcf.reshape(EPAD // 256, 1, 256)
    dst2d = dstf.reshape(EPAD // 128, 1, 128)

    b1_row = b1.reshape(1, 128)
    b2_row = b2.reshape(1, 128)
    b3_row = b3.reshape(1, 128)
    wrel1 = p1_Wrel.reshape(1, 128)
    wroot1 = p1_Wroot.reshape(1, 128)
    wrel2 = p2_Wrel.reshape(1, 128)
    wroot2 = p2_Wroot.reshape(1, 128)
    pb1 = p1_b.reshape(1, 1)
    pb2 = p2_b.reshape(1, 1)
    l1b_row = l1b.reshape(1, 128)
    l2w_pad = jnp.pad(l2W, ((0, 0), (0, 128 - C)))
    l2b_row = jnp.pad(l2b, (0, 128 - C)).reshape(1, 128)

    def stage(x_in, w, bias_row, s_row, m_row, wrel_row, wroot_row):
        indegp = _sc_scalar_scatter(m_row.reshape(PAD_N), srcf, dstf)
        a, dinv_row = _tc_mm_scale(x_in, w, s_row, m_row,
                                   indegp.reshape(NW, NBLK, 128))
        scp = _sc_feat_scatter(a, src2d, dst2d)
        h, v_row, t_row = _tc_post_conv(scp, a, dinv_row, bias_row,
                                        wrel_row, wroot_row, m_row)
        return h, v_row, t_row, a, dinv_row, scp

    def pool(v_row, t_row, m_row, pb, s1_row):
        aggp = _sc_scalar_scatter(v_row.reshape(PAD_N), srcf, dstf)
        score_row = _tc_score(aggp.reshape(NW, NBLK, 128), t_row, m_row, pb)
        kept_row, s_row = _tc_topk(score_row, s1_row, bf_row, m_row, fb)
        return score_row, kept_row, s_row

    zeros_row = jnp.zeros((NBLK, 128), _f32)

    h1, v1, t1, _, _, _ = stage(x_pad, W1, b1_row, m0_row, m0_row,
                                wrel1, wroot1)
    score1, kept1, s1 = pool(v1, t1, m0_row, pb1, zeros_row)

    h2, v2, t2, _, _, _ = stage(h1, W2, b2_row, s1, kept1, wrel2, wroot2)
    score2, kept2, s2 = pool(v2, t2, kept1, pb2, score1)

    indegp3 = _sc_scalar_scatter(kept2.reshape(PAD_N), srcf, dstf)
    a3, dinv3_row = _tc_mm_scale(h2, W3, s2, kept2,
                                 indegp3.reshape(NW, NBLK, 128))
    scp3 = _sc_feat_scatter(a3, src2d, dst2d)
    out = _tc_readout(scp3, a3, dinv3_row, b3_row, kept2, bf_row, l1W,
                      l1b_row, l2w_pad, l2b_row)
    return (out[:, :C], jnp.zeros((), _f32))


# speedup vs baseline: 1.5096x; 1.5096x over previous
import functools

import jax
import jax.numpy as jnp
from jax import lax
from jax.experimental import pallas as pl
from jax.experimental.pallas import tpu as pltpu
from jax.experimental.pallas import tpu_sc as plsc

N = 10000
E = 320000
B = 64
D = 128
C = 10
NBLK = 79
PAD_N = NBLK * 128
NW = 32
EPAD = 323584
EPW = EPAD // NW
CH = EPW // 128
CHA = 112
CHB = 2 * CH - CHA
CHMX = max(CHA, CHB)
ESTG = 2608
RPT = PAD_N // 16

_f32 = jnp.float32

_SC_MESH = plsc.VectorSubcoreMesh(core_axis_name="c", subcore_axis_name="s",
                                  num_cores=2, num_subcores=16)


def _sc_scalar_body(tab_hbm, src_hbm, dst_hbm, out_hbm, tab_v, src_v, dst_v,
                    acc_v):
    c = lax.axis_index("c")
    s = lax.axis_index("s")
    wid = s * 2 + c
    pltpu.sync_copy(tab_hbm, tab_v)
    pltpu.sync_copy(src_hbm.at[pl.ds(wid * EPW, EPW)], src_v)
    pltpu.sync_copy(dst_hbm.at[pl.ds(wid * EPW, EPW)], dst_v)

    def zero(i, carry):
        acc_v[pl.ds(i * 16, 16)] = jnp.zeros((16,), _f32)
        return carry

    lax.fori_loop(0, PAD_N // 16, zero, 0)

    def edge(i, carry):
        si = src_v[pl.ds(i * 16, 16)]
        di = dst_v[pl.ds(i * 16, 16)]
        vals = plsc.load_gather(tab_v, [si])
        plsc.addupdate_scatter(acc_v, [di], vals)
        return carry

    lax.fori_loop(0, EPW // 16, edge, 0)
    pltpu.sync_copy(acc_v, out_hbm.at[pl.ds(wid * PAD_N, PAD_N)])


_sc_scalar_scatter = pl.kernel(
    _sc_scalar_body,
    out_type=jax.ShapeDtypeStruct((NW * PAD_N,), _f32),
    mesh=_SC_MESH,
    scratch_types=[
        pltpu.VMEM((PAD_N,), _f32),
        pltpu.VMEM((EPW,), jnp.int32),
        pltpu.VMEM((EPW,), jnp.int32),
        pltpu.VMEM((PAD_N,), _f32),
    ],
    compiler_params=pltpu.CompilerParams(needs_layout_passes=False),
    name="sc_scalar_scatter",
)


def _sc_feat_body(a_hbm, src_hbm, dst_hbm, out_hbm, src_v, dst_v, rows0,
                  shacc, sem0):
    c = lax.axis_index("c")
    s = lax.axis_index("s")
    wid = s * 2 + c
    def zrow(k, carry):
        rows0[k // 8, pl.ds((k % 8) * 16, 16)] = jnp.zeros((16,), _f32)
        return carry

    lax.fori_loop(0, 128 * 8, zrow, 0)
    base = s * RPT
    for t in range(4):
        pltpu.sync_copy(rows0, shacc.at[pl.ds(base + t * 128, 128)])
    pltpu.sync_copy(rows0.at[pl.ds(0, RPT - 512)],
                    shacc.at[pl.ds(base + 512, RPT - 512)])
    plsc.subcore_barrier()

    nch = jnp.where(c == 0, CHA, CHB)
    off = c * (16 * CHA) + s * nch
    pltpu.sync_copy(src_hbm.at[pl.ds(off, CHMX)], src_v)
    pltpu.sync_copy(dst_hbm.at[pl.ds(off, CHMX)], dst_v)

    def chunk(j, carry):
        pltpu.async_copy(a_hbm.at[src_v.at[j, 0]], rows0, sem0).wait()
        pltpu.sync_copy(rows0, shacc.at[dst_v.at[j, 0]], add=True)
        return carry

    lax.fori_loop(0, nch, chunk, 0)

    plsc.subcore_barrier()
    pltpu.sync_copy(shacc.at[pl.ds(base, RPT)],
                    out_hbm.at[c].at[pl.ds(base, RPT)])


_sc_feat_scatter = pl.kernel(
    _sc_feat_body,
    out_type=jax.ShapeDtypeStruct((2, PAD_N, D), _f32),
    mesh=_SC_MESH,
    scratch_types=[
        pltpu.VMEM((CHMX, 1, 128), jnp.int32),
        pltpu.VMEM((CHMX, 1, 128), jnp.int32),
        pltpu.VMEM((128, D), _f32),
        pltpu.VMEM_SHARED((PAD_N, D), _f32),
        pltpu.SemaphoreType.DMA,
    ],
    compiler_params=pltpu.CompilerParams(needs_layout_passes=False),
    name="sc_feat_scatter",
)


def _tcol(row, eye):
    return lax.dot_general(eye, row, (((1,), (1,)), ((), ())),
                           preferred_element_type=_f32)


def _trow(col, eye):
    return lax.dot_general(col, eye, (((0,), (0,)), ((), ())),
                           preferred_element_type=_f32)


def _mm_scale_body(x_ref, w_ref, s_ref, m_ref, aggp_ref, a_ref, dinv_ref):
    eye = jnp.eye(128, dtype=_f32)
    w = w_ref[...]

    acc = lax.fori_loop(0, NW, lambda p, a: a + aggp_ref[p],
                        jnp.zeros((NBLK, 128), _f32))
    dinv = m_ref[...] * lax.rsqrt(acc + 1.0)
    dinv_ref[...] = dinv

    def blk(i, carry):
        xb = x_ref[pl.ds(i * 128, 128), :]
        s_col = _tcol(s_ref[pl.ds(i, 1), :], eye)
        d_col = _tcol(dinv_ref[pl.ds(i, 1), :], eye)
        xw = jnp.dot(xb * s_col, w, preferred_element_type=_f32)
        a_ref[pl.ds(i * 128, 128), :] = xw * d_col
        return carry

    lax.fori_loop(0, NBLK, blk, 0)


def _tc_mm_scale(x, w, s_row, m_row, aggp):
    return pl.pallas_call(
        _mm_scale_body,
        out_shape=(jax.ShapeDtypeStruct((PAD_N, D), _f32),
                   jax.ShapeDtypeStruct((NBLK, 128), _f32)),
    )(x, w, s_row, m_row, aggp)


def _post_conv_body(sc_ref, a_ref, dinv_ref, b_ref, wrel_ref, wroot_ref,
                    m_ref, h_ref, v_ref, t_ref):
    eye = jnp.eye(128, dtype=_f32)
    b = b_ref[...]
    wrel_c = _tcol(wrel_ref[...], eye)
    wroot_c = _tcol(wroot_ref[...], eye)

    def blk(i, carry):
        d_col = _tcol(dinv_ref[pl.ds(i, 1), :], eye)
        tot = (sc_ref[0, pl.ds(i * 128, 128), :]
               + sc_ref[1, pl.ds(i * 128, 128), :]
               + a_ref[pl.ds(i * 128, 128), :])
        hb = jnp.maximum(d_col * tot + b, 0.0)
        h_ref[pl.ds(i * 128, 128), :] = hb
        r_col = jnp.dot(hb, wrel_c, preferred_element_type=_f32)
        t_col = jnp.dot(hb, wroot_c, preferred_element_type=_f32)
        v_ref[pl.ds(i, 1), :] = _trow(r_col, eye) * m_ref[pl.ds(i, 1), :]
        t_ref[pl.ds(i, 1), :] = _trow(t_col, eye)
        return carry

    lax.fori_loop(0, NBLK, blk, 0)


def _tc_post_conv(sc, a, dinv_row, b_row, wrel_row, wroot_row, m_row):
    return pl.pallas_call(
        _post_conv_body,
        out_shape=(jax.ShapeDtypeStruct((PAD_N, D), _f32),
                   jax.ShapeDtypeStruct((NBLK, 128), _f32),
                   jax.ShapeDtypeStruct((NBLK, 128), _f32)),
    )(sc, a, dinv_row, b_row, wrel_row, wroot_row, m_row)


def _topk_body(aggp_ref, t_ref, pb_ref, s1_ref, bf_ref, mp_ref, fb_ref,
               kept_ref, srow_ref, score_ref, rank_ref):
    eye = jnp.eye(128, dtype=_f32)
    bf = bf_ref[...]
    mp = mp_ref[...]
    acc = lax.fori_loop(0, NW, lambda p, a: a + aggp_ref[p],
                        jnp.zeros((NBLK, 128), _f32))
    score_ref[...] = jnp.tanh(mp * acc + t_ref[...] + pb_ref[0, 0])
    s2_ref = score_ref

    kn = jnp.zeros((NBLK, 128), _f32)
    for g in range(B):
        cnt = jnp.sum(jnp.where(bf == g, mp, 0.0))
        kg = jnp.floor((cnt + 1.0) * 0.5)
        kn = kn + jnp.where(bf == g, kg, 0.0)

    rank_ref[...] = jnp.zeros((NBLK, 128), _f32)
    lane = lax.broadcasted_iota(jnp.int32, (1, 128), 1).astype(_f32)
    sub = lax.broadcasted_iota(jnp.int32, (128, 1), 0).astype(_f32)

    def jloop(j, carry):
        s2_jc = _tcol(s2_ref[pl.ds(j, 1), :], eye)
        s1_jc = _tcol(s1_ref[pl.ds(j, 1), :], eye)
        bf_jc = _tcol(bf_ref[pl.ds(j, 1), :], eye)
        mp_jc = _tcol(mp_ref[pl.ds(j, 1), :], eye)
        idx_j = sub + jnp.float32(128.0) * j.astype(_f32)
        fbj = fb_ref[j]
        fbj1 = fb_ref[j + 1]

        def iloop(i, carry2):
            fbi = fb_ref[i]
            fbi1 = fb_ref[i + 1]

            @pl.when(jnp.logical_and(fbj <= fbi1, fbj1 >= fbi))
            def _():
                s2_i = s2_ref[pl.ds(i, 1), :]
                s1_i = s1_ref[pl.ds(i, 1), :]
                bf_i = bf_ref[pl.ds(i, 1), :]
                idx_i = lane + jnp.float32(128.0) * i.astype(_f32)
                eq2 = s2_jc == s2_i
                beats = (s2_jc > s2_i) | (eq2 & ((s1_jc > s1_i) | (
                    (s1_jc == s1_i) & (idx_j < idx_i))))
                same = (bf_jc == bf_i) & (mp_jc > 0.0)
                contrib = jnp.sum(jnp.where(beats & same, 1.0, 0.0),
                                  axis=0, keepdims=True)
                rank_ref[pl.ds(i, 1), :] += contrib

            return carry2

        lax.fori_loop(0, NBLK, iloop, 0)
        return carry

    lax.fori_loop(0, NBLK, jloop, 0)

    kept = jnp.where((rank_ref[...] < kn) & (mp > 0.0) & (bf < float(B)),
                     1.0, 0.0)
    kept_ref[...] = kept
    srow_ref[...] = s2_ref[...] * kept


def _tc_topk(aggp, t_row, pb, s1_row, bf_row, mp_row, fb):
    return pl.pallas_call(
        _topk_body,
        in_specs=[
            pl.BlockSpec(memory_space=pltpu.VMEM),
            pl.BlockSpec(memory_space=pltpu.VMEM),
            pl.BlockSpec(memory_space=pltpu.SMEM),
            pl.BlockSpec(memory_space=pltpu.VMEM),
            pl.BlockSpec(memory_space=pltpu.VMEM),
            pl.BlockSpec(memory_space=pltpu.VMEM),
            pl.BlockSpec(memory_space=pltpu.SMEM),
        ],
        out_shape=(jax.ShapeDtypeStruct((NBLK, 128), _f32),
                   jax.ShapeDtypeStruct((NBLK, 128), _f32),
                   jax.ShapeDtypeStruct((NBLK, 128), _f32)),
        scratch_shapes=[pltpu.VMEM((NBLK, 128), _f32)],
    )(aggp, t_row, pb, s1_row, bf_row, mp_row, fb)


def _readout_body(sc_ref, a_ref, dinv_ref, b3_ref, m2_ref, bf_ref, l1w_ref,
                  l1b_ref, l2w_ref, l2b_ref, out_ref):
    eye = jnp.eye(128, dtype=_f32)
    b3 = b3_ref[...]
    g_row = lax.broadcasted_iota(jnp.int32, (1, B), 1).astype(_f32)

    def blk(i, carry):
        pooled, cnt = carry
        d_col = _tcol(dinv_ref[pl.ds(i, 1), :], eye)
        tot = (sc_ref[0, pl.ds(i * 128, 128), :]
               + sc_ref[1, pl.ds(i * 128, 128), :]
               + a_ref[pl.ds(i * 128, 128), :])
        hb = jnp.maximum(d_col * tot + b3, 0.0)
        m_col = _tcol(m2_ref[pl.ds(i, 1), :], eye)
        b_col = _tcol(bf_ref[pl.ds(i, 1), :], eye)
        onehot = jnp.where(b_col == g_row, 1.0, 0.0)
        pooled = pooled + lax.dot_general(onehot, hb * m_col,
                                          (((0,), (0,)), ((), ())),
                                          preferred_element_type=_f32)
        cnt = cnt + lax.dot_general(onehot, m_col, (((0,), (0,)), ((), ())),
                                    preferred_element_type=_f32)
        return pooled, cnt

    pooled, cnt = lax.fori_loop(
        0, NBLK, blk,
        (jnp.zeros((B, 128), _f32), jnp.zeros((B, 1), _f32)))
    pooled = pooled / jnp.maximum(cnt, 1.0)
    z = jnp.maximum(jnp.dot(pooled, l1w_ref[...],
                            preferred_element_type=_f32) + l1b_ref[...], 0.0)
    logits = jnp.dot(z, l2w_ref[...], preferred_element_type=_f32) \
        + l2b_ref[...]
    valid = lax.broadcasted_iota(jnp.int32, (1, 128), 1) < C
    neg = jnp.where(valid, logits, -jnp.inf)
    mx = jnp.max(neg, axis=1, keepdims=True)
    ez = jnp.where(valid, jnp.exp(logits - mx), 0.0)
    lse = jnp.log(jnp.sum(ez, axis=1, keepdims=True))
    out_ref[...] = logits - mx - lse


def _tc_readout(sc, a, dinv_row, b3_row, m2_row, bf_row, l1w, l1b_row,
                l2w_pad, l2b_row):
    return pl.pallas_call(
        _readout_body,
        out_shape=jax.ShapeDtypeStruct((B, 128), _f32),
    )(sc, a, dinv_row, b3_row, m2_row, bf_row, l1w, l1b_row, l2w_pad,
      l2b_row)


def kernel(x, edge_index, batch, W1, b1, p1_Wrel, p1_Wroot, p1_b, W2, b2,
           p2_Wrel, p2_Wroot, p2_b, W3, b3, l1W, l1b, l2W, l2b):
    n = x.shape[0]

    x_pad = jnp.pad(x, ((0, PAD_N - n), (0, 0)))
    batch_pad = jnp.pad(batch, (0, PAD_N - n), constant_values=B)
    bf_row = batch_pad.astype(_f32).reshape(NBLK, 128)
    fb = jnp.concatenate(
        [batch_pad.reshape(NBLK, 128)[:, 0],
         jnp.full((1,), B, jnp.int32)])
    m0_row = (jnp.arange(PAD_N, dtype=jnp.int32) < n).astype(_f32) \
        .reshape(NBLK, 128)

    srcf = jnp.pad(edge_index[0], (0, EPAD - E), constant_values=n)
    dstf = jnp.pad(edge_index[1], (0, EPAD - E), constant_values=n)
    src2d = jnp.pad(srcf, (0, ESTG * 128 - EPAD),
                    constant_values=n).reshape(ESTG, 1, 128)
    dst2d = jnp.pad(dstf, (0, ESTG * 128 - EPAD),
                    constant_values=n).reshape(ESTG, 1, 128)

    b1_row = b1.reshape(1, 128)
    b2_row = b2.reshape(1, 128)
    b3_row = b3.reshape(1, 128)
    wrel1 = p1_Wrel.reshape(1, 128)
    wroot1 = p1_Wroot.reshape(1, 128)
    wrel2 = p2_Wrel.reshape(1, 128)
    wroot2 = p2_Wroot.reshape(1, 128)
    pb1 = p1_b.reshape(1, 1)
    pb2 = p2_b.reshape(1, 1)
    l1b_row = l1b.reshape(1, 128)
    l2w_pad = jnp.pad(l2W, ((0, 0), (0, 128 - C)))
    l2b_row = jnp.pad(l2b, (0, 128 - C)).reshape(1, 128)

    def stage(x_in, w, bias_row, s_row, m_row, wrel_row, wroot_row):
        indegp = _sc_scalar_scatter(m_row.reshape(PAD_N), srcf, dstf)
        a, dinv_row = _tc_mm_scale(x_in, w, s_row, m_row,
                                   indegp.reshape(NW, NBLK, 128))
        scp = _sc_feat_scatter(a, src2d, dst2d)
        h, v_row, t_row = _tc_post_conv(scp, a, dinv_row, bias_row,
                                        wrel_row, wroot_row, m_row)
        return h, v_row, t_row, a, dinv_row, scp

    def pool(v_row, t_row, m_row, pb, s1_row):
        aggp = _sc_scalar_scatter(v_row.reshape(PAD_N), srcf, dstf)
        kept_row, s_row, score_row = _tc_topk(
            aggp.reshape(NW, NBLK, 128), t_row, pb, s1_row, bf_row, m_row, fb)
        return score_row, kept_row, s_row

    zeros_row = jnp.zeros((NBLK, 128), _f32)

    h1, v1, t1, _, _, _ = stage(x_pad, W1, b1_row, m0_row, m0_row,
                                wrel1, wroot1)
    score1, kept1, s1 = pool(v1, t1, m0_row, pb1, zeros_row)

    h2, v2, t2, _, _, _ = stage(h1, W2, b2_row, s1, kept1, wrel2, wroot2)
    score2, kept2, s2 = pool(v2, t2, kept1, pb2, score1)

    indegp3 = _sc_scalar_scatter(kept2.reshape(PAD_N), srcf, dstf)
    a3, dinv3_row = _tc_mm_scale(h2, W3, s2, kept2,
                                 indegp3.reshape(NW, NBLK, 128))
    scp3 = _sc_feat_scatter(a3, src2d, dst2d)
    out = _tc_readout(scp3, a3, dinv3_row, b3_row, kept2, bf_row, l1W,
                      l1b_row, l2w_pad, l2b_row)
    return (out[:, :C], jnp.zeros((), _f32))
